# baseline (speedup 1.0000x reference)
import functools

import jax
import jax.numpy as jnp
from jax import lax
from jax.experimental import pallas as pl
from jax.experimental.pallas import tpu as pltpu
from jax.experimental.pallas import tpu_sc as plsc

_ALPHA = 4
_NBUF = 2


def _make_sc_gather(B, T, C, H, W, dtype):
    S = T // _ALPHA
    mesh = plsc.VectorSubcoreMesh(core_axis_name="c", subcore_axis_name="s")

    @functools.partial(
        pl.kernel,
        out_type=jax.ShapeDtypeStruct((B, S, C, H, W), dtype),
        mesh=mesh,
        scratch_types=[pltpu.VMEM((_NBUF, H, W), dtype)]
        + [pltpu.SemaphoreType.DMA] * (2 * _NBUF),
    )
    def k(src_hbm, slow_hbm, buf, *sems):
        isem, osem = sems[:_NBUF], sems[_NBUF:]
        wid = lax.axis_index("s") * 2 + lax.axis_index("c")
        b = wid // S
        j = wid % S
        t_src = (j * (T - 1)) // (S - 1)

        ins, outs = [None] * C, [None] * C

        def start_in(i):
            ins[i] = pltpu.make_async_copy(
                src_hbm.at[b, t_src, i], buf.at[i % _NBUF], isem[i % _NBUF])
            ins[i].start()

        def start_out(i):
            outs[i] = pltpu.make_async_copy(
                buf.at[i % _NBUF], slow_hbm.at[b, j, i], osem[i % _NBUF])
            outs[i].start()

        start_in(0)
        for i in range(C):
            if i + 1 < C:
                if i + 1 >= _NBUF:
                    outs[i + 1 - _NBUF].wait()
                start_in(i + 1)
            ins[i].wait()
            start_out(i)
        for i in range(max(0, C - _NBUF), C):
            outs[i].wait()

    return k


def kernel(frames):
    B, T, C, H, W = frames.shape
    slow = _make_sc_gather(B, T, C, H, W, frames.dtype)(frames)
    return slow, frames


# speedup vs baseline: 1.1007x; 1.1007x over previous
import jax
import jax.numpy as jnp
from jax.experimental import pallas as pl

_ALPHA = 4


def kernel(frames):
    B, T, C, H, W = frames.shape
    S = T // _ALPHA

    def in_map(b, j):
        return (b, (j * (T - 1)) // (S - 1), 0, 0, 0)

    def body(in_ref, out_ref):
        out_ref[...] = in_ref[...]

    slow = pl.pallas_call(
        body,
        grid=(B, S),
        in_specs=[pl.BlockSpec((1, 1, C, H, W), in_map)],
        out_specs=pl.BlockSpec((1, 1, C, H, W), lambda b, j: (b, j, 0, 0, 0)),
        out_shape=jax.ShapeDtypeStruct((B, S, C, H, W), frames.dtype),
    )(frames)
    return slow, frames


# speedup vs baseline: 1.2645x; 1.1488x over previous
import jax
import jax.numpy as jnp
from jax.experimental import pallas as pl

_ALPHA = 4


def kernel(frames):
    B, T, C, H, W = frames.shape
    S = T // _ALPHA

    def in_map(j):
        return (0, (j * (T - 1)) // (S - 1), 0, 0, 0)

    def body(in_ref, out_ref):
        out_ref[...] = in_ref[...]

    slow = pl.pallas_call(
        body,
        grid=(S,),
        in_specs=[pl.BlockSpec((B, 1, C, H, W), in_map)],
        out_specs=pl.BlockSpec((B, 1, C, H, W), lambda j: (0, j, 0, 0, 0)),
        out_shape=jax.ShapeDtypeStruct((B, S, C, H, W), frames.dtype),
    )(frames)
    return slow, frames


# speedup vs baseline: 1.3843x; 1.0947x over previous
import jax
import jax.numpy as jnp
from jax.experimental import pallas as pl

_ALPHA = 4


def kernel(frames):
    B, T, C, H, W = frames.shape
    S = T // _ALPHA

    def body(in_ref, slow_ref, fast_ref):
        fast_ref[...] = in_ref[...]
        g = pl.program_id(1)
        toff = (g * (T - 1)) // (S - 1) - _ALPHA * g
        slow_ref[0, 0] = in_ref[0, pl.ds(toff, 1)][0]

    slow, fast = pl.pallas_call(
        body,
        grid=(B, S),
        in_specs=[pl.BlockSpec((1, _ALPHA, C, H, W), lambda b, g: (b, g, 0, 0, 0))],
        out_specs=[
            pl.BlockSpec((1, 1, C, H, W), lambda b, g: (b, g, 0, 0, 0)),
            pl.BlockSpec((1, _ALPHA, C, H, W), lambda b, g: (b, g, 0, 0, 0)),
        ],
        out_shape=[
            jax.ShapeDtypeStruct((B, S, C, H, W), frames.dtype),
            jax.ShapeDtypeStruct((B, T, C, H, W), frames.dtype),
        ],
    )(frames)
    return slow, fast
